# baseline (speedup 1.0000x reference)
import functools

import jax
import jax.numpy as jnp
from jax import lax
from jax.experimental import pallas as pl
from jax.experimental.pallas import tpu as pltpu
from jax.experimental.pallas import tpu_sc as plsc

_B, _T, _C = 16, 4096, 512
_BLANK = _C - 1
_L = 16
_NCHUNK = _T // _L

_LANES = 128


_RB, _TB = 8, 256


def _argmax_body(x_ref, o_ref):
    bv = x_ref[:, :, 0:_LANES]
    bk = jnp.zeros(bv.shape, jnp.float32)
    for k in range(1, _C // _LANES):
        xk = x_ref[:, :, k * _LANES:(k + 1) * _LANES]
        m = xk > bv
        bv = jnp.where(m, xk, bv)
        bk = jnp.where(m, float(k), bk)
    mx = jnp.max(bv, axis=-1, keepdims=True)
    lane = lax.broadcasted_iota(jnp.int32, (1, 1, _LANES), 2).astype(jnp.float32)
    full = bk * float(_LANES) + lane
    idx = jnp.min(jnp.where(bv == mx, full, float(_C)), axis=-1)
    o_ref[...] = idx.astype(jnp.int32)


def _tc_argmax(y_pred):
    return pl.pallas_call(
        _argmax_body,
        grid=(_B // _RB, _T // _TB),
        in_specs=[pl.BlockSpec((_RB, _TB, _C), lambda i, j: (i, j, 0))],
        out_specs=pl.BlockSpec((_RB, _TB), lambda i, j: (i, j)),
        out_shape=jax.ShapeDtypeStruct((_B, _T), jnp.int32),
    )(y_pred)


def _sc_decode(paths, neg_row):
    nc, ns = 2, 16
    mesh = plsc.VectorSubcoreMesh(core_axis_name="c", subcore_axis_name="s",
                                  num_cores=nc, num_subcores=ns)

    @functools.partial(
        pl.kernel,
        out_type=jax.ShapeDtypeStruct((_B, _T), jnp.int32),
        mesh=mesh,
        compiler_params=pltpu.CompilerParams(needs_layout_passes=False),
        scratch_types=[
            pltpu.VMEM((_T,), jnp.int32),
            pltpu.VMEM((_T,), jnp.int32),
        ],
    )
    def k(paths_hbm, neg_hbm, out_hbm, path_v, out_v):
        wid = lax.axis_index("s") * nc + lax.axis_index("c")

        @pl.when(wid < _B)
        def _():
            pltpu.sync_copy(neg_hbm, out_v)
            pltpu.sync_copy(paths_hbm.at[wid], path_v)
            lane = lax.broadcasted_iota(jnp.int32, (_L,), 0)

            shift_idx = jnp.maximum(lane - 1, 0)
            last_idx = jnp.full((_L,), _L - 1, jnp.int32)

            def body(i, carry):
                cnt_vec, last = carry
                v = path_v[pl.ds(i * _L, _L)]
                shifted = v.at[shift_idx].get(mode="promise_in_bounds")
                prev = jnp.where(lane == 0, last, shifted)
                valid = (v != _BLANK) & (v != prev)
                csum = plsc.cumsum(valid.astype(jnp.int32))
                pos = cnt_vec + csum - 1
                plsc.store_scatter(out_v, [pos], v, mask=valid)
                cnt_vec = cnt_vec + plsc.all_reduce_population_count(valid)
                return cnt_vec, v.at[last_idx].get(mode="promise_in_bounds")

            lax.fori_loop(
                0, _NCHUNK, body,
                (jnp.zeros((_L,), jnp.int32), jnp.full((_L,), -1, jnp.int32)),
                unroll=4)
            pltpu.sync_copy(out_v, out_hbm.at[wid])

    return k(paths, neg_row)


def kernel(y_pred):
    paths = _tc_argmax(y_pred)
    neg_row = jnp.full((_T,), -1, jnp.int32)
    decoded = _sc_decode(paths, neg_row)
    return decoded.reshape(-1, 1)


# speedup vs baseline: 1.1414x; 1.1414x over previous
import functools

import jax
import jax.numpy as jnp
from jax import lax
from jax.experimental import pallas as pl
from jax.experimental.pallas import tpu as pltpu
from jax.experimental.pallas import tpu_sc as plsc

_B, _T, _C = 16, 4096, 512
_BLANK = _C - 1
_L = 16
_NCHUNK = _T // _L

_LANES = 128


_RB, _TB = 8, 512


def _argmax_body(x_ref, o_ref):
    bv = x_ref[:, :, 0:_LANES]
    bk = jnp.zeros(bv.shape, jnp.float32)
    for k in range(1, _C // _LANES):
        xk = x_ref[:, :, k * _LANES:(k + 1) * _LANES]
        m = xk > bv
        bv = jnp.where(m, xk, bv)
        bk = jnp.where(m, float(k), bk)
    mx = jnp.max(bv, axis=-1, keepdims=True)
    lane = lax.broadcasted_iota(jnp.int32, (1, 1, _LANES), 2).astype(jnp.float32)
    full = bk * float(_LANES) + lane
    idx = jnp.min(jnp.where(bv == mx, full, float(_C)), axis=-1)
    o_ref[...] = idx.astype(jnp.int32)


def _tc_argmax(y_pred):
    return pl.pallas_call(
        _argmax_body,
        grid=(_B // _RB, _T // _TB),
        in_specs=[pl.BlockSpec((_RB, _TB, _C), lambda i, j: (i, j, 0))],
        out_specs=pl.BlockSpec((_RB, _TB), lambda i, j: (i, j)),
        out_shape=jax.ShapeDtypeStruct((_B, _T), jnp.int32),
    )(y_pred)


def _sc_decode(paths):
    nc, ns = 2, 8
    mesh = plsc.VectorSubcoreMesh(core_axis_name="c", subcore_axis_name="s",
                                  num_cores=nc, num_subcores=ns)

    @functools.partial(
        pl.kernel,
        out_type=jax.ShapeDtypeStruct((_B, _T), jnp.int32),
        mesh=mesh,
        compiler_params=pltpu.CompilerParams(needs_layout_passes=False),
        scratch_types=[
            pltpu.VMEM((_T,), jnp.int32),
            pltpu.VMEM((_T,), jnp.int32),
        ],
    )
    def k(paths_hbm, out_hbm, path_v, out_v):
        wid = lax.axis_index("s") * nc + lax.axis_index("c")
        pltpu.sync_copy(paths_hbm.at[wid], path_v)
        lane = lax.broadcasted_iota(jnp.int32, (_L,), 0)

        shift_idx = jnp.maximum(lane - 1, 0)
        last_idx = jnp.full((_L,), _L - 1, jnp.int32)
        neg = jnp.full((_L,), -1, jnp.int32)

        def body(i, carry):
            cnt_vec, last = carry
            out_v[pl.ds(i * _L, _L)] = neg
            v = path_v[pl.ds(i * _L, _L)]
            shifted = v.at[shift_idx].get(mode="promise_in_bounds")
            prev = jnp.where(lane == 0, last, shifted)
            valid = (v != _BLANK) & (v != prev)
            csum = plsc.cumsum(valid.astype(jnp.int32))
            pos = cnt_vec + csum - 1
            plsc.store_scatter(out_v, [pos], v, mask=valid)
            cnt_vec = cnt_vec + plsc.all_reduce_population_count(valid)
            return cnt_vec, v.at[last_idx].get(mode="promise_in_bounds")

        lax.fori_loop(
            0, _NCHUNK, body,
            (jnp.zeros((_L,), jnp.int32), jnp.full((_L,), -1, jnp.int32)),
            unroll=4)
        pltpu.sync_copy(out_v, out_hbm.at[wid])

    return k(paths)


def kernel(y_pred):
    paths = _tc_argmax(y_pred)
    decoded = _sc_decode(paths)
    return decoded.reshape(-1, 1)


# speedup vs baseline: 1.1453x; 1.0034x over previous
import functools

import jax
import jax.numpy as jnp
from jax import lax
from jax.experimental import pallas as pl
from jax.experimental.pallas import tpu as pltpu
from jax.experimental.pallas import tpu_sc as plsc

_B, _T, _C = 16, 4096, 512
_BLANK = _C - 1
_L = 16
_NCHUNK = _T // _L

_LANES = 128


_RB, _TB = 8, 512


def _argmax_body(x_ref, o_ref):
    bv = x_ref[:, :, 0:_LANES]
    bk = jnp.zeros(bv.shape, jnp.float32)
    for k in range(1, _C // _LANES):
        xk = x_ref[:, :, k * _LANES:(k + 1) * _LANES]
        m = xk > bv
        bv = jnp.where(m, xk, bv)
        bk = jnp.where(m, float(k), bk)
    mx = jnp.max(bv, axis=-1, keepdims=True)
    lane = lax.broadcasted_iota(jnp.int32, (1, 1, _LANES), 2).astype(jnp.float32)
    full = bk * float(_LANES) + lane
    idx = jnp.min(jnp.where(bv == mx, full, float(_C)), axis=-1)
    o_ref[...] = idx.astype(jnp.int32)


def _tc_argmax(y_pred):
    return pl.pallas_call(
        _argmax_body,
        grid=(_B // _RB, _T // _TB),
        in_specs=[pl.BlockSpec((_RB, _TB, _C), lambda i, j: (i, j, 0))],
        out_specs=pl.BlockSpec((_RB, _TB), lambda i, j: (i, j)),
        out_shape=jax.ShapeDtypeStruct((_B, _T), jnp.int32),
        compiler_params=pltpu.CompilerParams(
            dimension_semantics=("parallel", "parallel")),
    )(y_pred)


def _sc_decode(paths):
    nc, ns = 2, 8
    mesh = plsc.VectorSubcoreMesh(core_axis_name="c", subcore_axis_name="s",
                                  num_cores=nc, num_subcores=ns)

    @functools.partial(
        pl.kernel,
        out_type=jax.ShapeDtypeStruct((_B, _T), jnp.int32),
        mesh=mesh,
        compiler_params=pltpu.CompilerParams(needs_layout_passes=False),
        scratch_types=[
            pltpu.VMEM((_T,), jnp.int32),
            pltpu.VMEM((_T,), jnp.int32),
        ],
    )
    def k(paths_hbm, out_hbm, path_v, out_v):
        wid = lax.axis_index("s") * nc + lax.axis_index("c")
        pltpu.sync_copy(paths_hbm.at[wid], path_v)
        lane = lax.broadcasted_iota(jnp.int32, (_L,), 0)

        shift_idx = jnp.maximum(lane - 1, 0)
        last_idx = jnp.full((_L,), _L - 1, jnp.int32)
        neg = jnp.full((_L,), -1, jnp.int32)

        def body(i, carry):
            cnt_vec, last = carry
            out_v[pl.ds(i * _L, _L)] = neg
            v = path_v[pl.ds(i * _L, _L)]
            shifted = v.at[shift_idx].get(mode="promise_in_bounds")
            prev = jnp.where(lane == 0, last, shifted)
            valid = (v != _BLANK) & (v != prev)
            csum = plsc.cumsum(valid.astype(jnp.int32))
            pos = cnt_vec + csum - 1
            plsc.store_scatter(out_v, [pos], v, mask=valid)
            cnt_vec = cnt_vec + plsc.all_reduce_population_count(valid)
            return cnt_vec, v.at[last_idx].get(mode="promise_in_bounds")

        lax.fori_loop(
            0, _NCHUNK, body,
            (jnp.zeros((_L,), jnp.int32), jnp.full((_L,), -1, jnp.int32)),
            unroll=4)
        pltpu.sync_copy(out_v, out_hbm.at[wid])

    return k(paths)


def kernel(y_pred):
    paths = _tc_argmax(y_pred)
    decoded = _sc_decode(paths)
    return decoded.reshape(-1, 1)
